# baseline (speedup 1.0000x reference)
import jax
import jax.numpy as jnp
from jax.experimental import pallas as pl
from jax.experimental.pallas import tpu as pltpu

_N_CHUNKS = 8


def _copy_body(pe_ref, out_ref, *sems):
    rows = pe_ref.shape[0]
    chunk = rows // _N_CHUNKS
    copies = []
    for c in range(_N_CHUNKS):
        cp = pltpu.make_async_copy(
            pe_ref.at[pl.ds(c * chunk, chunk)],
            out_ref.at[pl.ds(c * chunk, chunk)],
            sems[c],
        )
        cp.start()
        copies.append(cp)
    for cp in copies:
        cp.wait()


def kernel(x, pe):
    seq_len = x.shape[1]
    hidden = pe.shape[1]
    out = pl.pallas_call(
        _copy_body,
        in_specs=[pl.BlockSpec(memory_space=pl.ANY)],
        out_specs=pl.BlockSpec(memory_space=pl.ANY),
        out_shape=jax.ShapeDtypeStruct((seq_len, hidden), pe.dtype),
        scratch_shapes=[pltpu.SemaphoreType.DMA] * _N_CHUNKS,
    )(pe[:seq_len])
    return out[None, ...]


# speedup vs baseline: 48.1553x; 48.1553x over previous
import jax
import jax.numpy as jnp
from jax.experimental import pallas as pl
from jax.experimental.pallas import tpu as pltpu

_N_CHUNKS = 8


def _copy_body(pe_ref, out_ref, buf, *sems):
    rows = pe_ref.shape[0]
    chunk = rows // _N_CHUNKS
    isems = sems[:_N_CHUNKS]
    osems = sems[_N_CHUNKS:]
    ins = []
    for c in range(_N_CHUNKS):
        cp = pltpu.make_async_copy(
            pe_ref.at[pl.ds(c * chunk, chunk)], buf.at[c], isems[c]
        )
        cp.start()
        ins.append(cp)
    outs = []
    for c in range(_N_CHUNKS):
        ins[c].wait()
        cp = pltpu.make_async_copy(
            buf.at[c], out_ref.at[pl.ds(c * chunk, chunk)], osems[c]
        )
        cp.start()
        outs.append(cp)
    for cp in outs:
        cp.wait()


def kernel(x, pe):
    seq_len = x.shape[1]
    hidden = pe.shape[1]
    chunk = seq_len // _N_CHUNKS
    out = pl.pallas_call(
        _copy_body,
        in_specs=[pl.BlockSpec(memory_space=pl.ANY)],
        out_specs=pl.BlockSpec(memory_space=pl.ANY),
        out_shape=jax.ShapeDtypeStruct((seq_len, hidden), pe.dtype),
        scratch_shapes=(
            [pltpu.VMEM((_N_CHUNKS, chunk, hidden), pe.dtype)]
            + [pltpu.SemaphoreType.DMA] * (2 * _N_CHUNKS)
        ),
    )(pe[:seq_len])
    return out[None, ...]
